# baseline (speedup 1.0000x reference)
import jax
import jax.numpy as jnp
from jax import lax
from jax.experimental import pallas as pl
from jax.experimental.pallas import tpu as pltpu
from jax.experimental.pallas import tpu_sc as plsc

_NC, _NS = 2, 16
_NW = _NC * _NS


def _make_body(B, L, D, bpw):
    def body(x_hbm, tbl_hbm, out_hbm, buf):
        wid = lax.axis_index("s") * _NC + lax.axis_index("c")
        base = wid * bpw
        pltpu.sync_copy(tbl_hbm, buf.at[:, pl.ds(D, D)])

        def step(i, carry):
            pltpu.sync_copy(x_hbm.at[base + i], buf.at[:, pl.ds(0, D)])
            pltpu.sync_copy(buf, out_hbm.at[base + i])
            return carry

        lax.fori_loop(0, bpw, step, 0)

    return body


def kernel(x, position_table):
    B, L, D = x.shape
    tbl = position_table[:L]
    bpw = B // _NW
    mesh = plsc.VectorSubcoreMesh(core_axis_name="c", subcore_axis_name="s")
    f = pl.kernel(
        _make_body(B, L, D, bpw),
        out_type=jax.ShapeDtypeStruct((B, L, 2 * D), x.dtype),
        mesh=mesh,
        scratch_types=[pltpu.VMEM((L, 2 * D), x.dtype)],
        compiler_params=pltpu.CompilerParams(use_tc_tiling_on_sc=False),
    )
    return f(x, tbl)


# speedup vs baseline: 1.1487x; 1.1487x over previous
import jax
import jax.numpy as jnp
from jax import lax
from jax.experimental import pallas as pl
from jax.experimental.pallas import tpu as pltpu
from jax.experimental.pallas import tpu_sc as plsc

_NC, _NS = 2, 16
_NW = _NC * _NS
_NBUF = 4


def _make_body(B, L, D, bpw):
    nchunks = bpw // _NBUF

    def body(x_hbm, tbl_hbm, out_hbm, *rest):
        bufs = rest[:_NBUF]
        sins = rest[_NBUF:2 * _NBUF]
        souts = rest[2 * _NBUF:3 * _NBUF]
        wid = lax.axis_index("s") * _NC + lax.axis_index("c")
        base = wid * bpw

        def in_copy(n, j):
            return pltpu.make_async_copy(
                x_hbm.at[base + j], bufs[n].at[:, pl.ds(0, D)], sins[n])

        def out_copy(n, j):
            return pltpu.make_async_copy(bufs[n], out_hbm.at[base + j], souts[n])

        for n in range(_NBUF):
            pltpu.sync_copy(tbl_hbm, bufs[n].at[:, pl.ds(D, D)])
        for n in range(_NBUF):
            in_copy(n, n).start()

        def chunk(c, carry):
            j0 = c * _NBUF
            for n in range(_NBUF):
                in_copy(n, j0 + n).wait()
                out_copy(n, j0 + n).start()
            for n in range(_NBUF):
                out_copy(n, j0 + n).wait()
                in_copy(n, j0 + n + _NBUF).start()
            return carry

        lax.fori_loop(0, nchunks - 1, chunk, 0)

        j0 = (nchunks - 1) * _NBUF
        for n in range(_NBUF):
            in_copy(n, j0 + n).wait()
            out_copy(n, j0 + n).start()
        for n in range(_NBUF):
            out_copy(n, j0 + n).wait()

    return body


def kernel(x, position_table):
    B, L, D = x.shape
    tbl = position_table[:L]
    bpw = B // _NW
    mesh = plsc.VectorSubcoreMesh(core_axis_name="c", subcore_axis_name="s")
    f = pl.kernel(
        _make_body(B, L, D, bpw),
        out_type=jax.ShapeDtypeStruct((B, L, 2 * D), x.dtype),
        mesh=mesh,
        scratch_types=(
            [pltpu.VMEM((L, 2 * D), x.dtype) for _ in range(_NBUF)]
            + [pltpu.SemaphoreType.DMA for _ in range(2 * _NBUF)]
        ),
        compiler_params=pltpu.CompilerParams(use_tc_tiling_on_sc=False),
    )
    return f(x, tbl)


# speedup vs baseline: 1.3984x; 1.2173x over previous
import jax
import jax.numpy as jnp
from jax import lax
from jax.experimental import pallas as pl
from jax.experimental.pallas import tpu as pltpu
from jax.experimental.pallas import tpu_sc as plsc

_NC, _NS = 2, 16
_NW = _NC * _NS
_NBUF = 2


def _make_body(B, L, D, bpw):
    nchunks = bpw // _NBUF

    def body(x_hbm, tbl_hbm, out_hbm, *rest):
        xvs = rest[:_NBUF]
        bufs = rest[_NBUF:2 * _NBUF]
        sins = rest[2 * _NBUF:3 * _NBUF]
        souts = rest[3 * _NBUF:4 * _NBUF]
        wid = lax.axis_index("s") * _NC + lax.axis_index("c")
        base = wid * bpw

        def in_copy(n, j):
            return pltpu.make_async_copy(x_hbm.at[base + j], xvs[n], sins[n])

        def out_copy(n, j):
            return pltpu.make_async_copy(bufs[n], out_hbm.at[base + j], souts[n])

        def assemble(n):
            def row(r, carry):
                for c in range(D // 16):
                    bufs[n][r, pl.ds(c * 16, 16)] = xvs[n][r, pl.ds(c * 16, 16)]
                return carry
            lax.fori_loop(0, L, row, 0)

        for n in range(_NBUF):
            pltpu.sync_copy(tbl_hbm, bufs[n])
        for n in range(_NBUF):
            in_copy(n, n).start()

        for n in range(_NBUF):
            in_copy(n, n).wait()
            assemble(n)
            out_copy(n, n).start()
            in_copy(n, n + _NBUF).start()

        def chunk(c, carry):
            j0 = c * _NBUF
            for n in range(_NBUF):
                in_copy(n, j0 + n).wait()
                out_copy(n, j0 + n - _NBUF).wait()
                assemble(n)
                out_copy(n, j0 + n).start()
                in_copy(n, j0 + n + _NBUF).start()
            return carry

        lax.fori_loop(1, nchunks - 1, chunk, 0)

        j0 = (nchunks - 1) * _NBUF
        for n in range(_NBUF):
            in_copy(n, j0 + n).wait()
            out_copy(n, j0 + n - _NBUF).wait()
            assemble(n)
            out_copy(n, j0 + n).start()
        for n in range(_NBUF):
            out_copy(n, j0 + n).wait()

    return body


def kernel(x, position_table):
    B, L, D = x.shape
    tbl = jnp.concatenate(
        [jnp.zeros((L, D), x.dtype), position_table[:L]], axis=-1)
    bpw = B // _NW
    mesh = plsc.VectorSubcoreMesh(core_axis_name="c", subcore_axis_name="s")
    f = pl.kernel(
        _make_body(B, L, D, bpw),
        out_type=jax.ShapeDtypeStruct((B, L, 2 * D), x.dtype),
        mesh=mesh,
        scratch_types=(
            [pltpu.VMEM((L, D), x.dtype) for _ in range(_NBUF)]
            + [pltpu.VMEM((L, 2 * D), x.dtype) for _ in range(_NBUF)]
            + [pltpu.SemaphoreType.DMA for _ in range(2 * _NBUF)]
        ),
        compiler_params=pltpu.CompilerParams(use_tc_tiling_on_sc=True),
    )
    return f(x, tbl)
